# baseline (speedup 1.0000x reference)
import functools

import jax
import jax.numpy as jnp
from jax import lax
from jax.experimental import pallas as pl
from jax.experimental.pallas import tpu as pltpu

_UNROLL = 16
_T_TILE = 1024


def _round_up(x, m):
    return (x + m - 1) // m * m


def _gather_kernel(idx_ref, table_hbm, out_ref, tab_vmem, sem, *, t_tile, unroll):
    t = pl.program_id(1)

    @pl.when(t == 0)
    def _load_table_half():
        dd = pl.program_id(0)
        cp = pltpu.make_async_copy(
            table_hbm.at[:, pl.ds(dd, 1), :], tab_vmem, sem)
        cp.start()
        cp.wait()

    base = t * t_tile

    def chunk(cb, carry):
        t0 = cb * unroll
        for u in range(unroll):
            loc = t0 + u
            row = idx_ref[base + loc]
            out_ref[loc, 0] = tab_vmem[row, 0]
        return carry

    lax.fori_loop(0, t_tile // unroll, chunk, 0)


def kernel(indices, table):
    b, s = indices.shape
    v, d = table.shape
    n_tok = b * s
    d_half = d // 2

    flat_idx = jnp.clip(indices.reshape(-1).astype(jnp.int32), 0, v - 1)

    t_tile = min(_T_TILE, _round_up(n_tok, _UNROLL))
    n_pad = _round_up(n_tok, t_tile)
    if n_pad != n_tok:
        flat_idx = jnp.pad(flat_idx, (0, n_pad - n_tok))
    n_tiles = n_pad // t_tile

    table_r = table.reshape(v, 2, d_half)

    grid_spec = pltpu.PrefetchScalarGridSpec(
        num_scalar_prefetch=1,
        grid=(2, n_tiles),
        in_specs=[pl.BlockSpec(memory_space=pl.ANY)],
        out_specs=pl.BlockSpec(
            (t_tile, 1, d_half), lambda dd, t, idx: (dd * n_tiles + t, 0, 0)),
        scratch_shapes=[
            pltpu.VMEM((v, 1, d_half), table.dtype),
            pltpu.SemaphoreType.DMA,
        ],
    )

    table_half_bytes = v * d_half * jnp.dtype(table.dtype).itemsize
    tile_bytes = t_tile * d_half * jnp.dtype(table.dtype).itemsize
    vmem_limit = int(min(table_half_bytes + 4 * tile_bytes + (8 << 20), 62 << 20))

    halves = pl.pallas_call(
        functools.partial(_gather_kernel, t_tile=t_tile, unroll=_UNROLL),
        out_shape=jax.ShapeDtypeStruct((2 * n_pad, 1, d_half), table.dtype),
        grid_spec=grid_spec,
        compiler_params=pltpu.CompilerParams(
            dimension_semantics=("parallel", "arbitrary"),
            vmem_limit_bytes=vmem_limit,
        ),
    )(flat_idx, table_r)

    halves = halves.reshape(2, n_pad, d_half)
    out = jnp.concatenate([halves[0, :n_tok], halves[1, :n_tok]], axis=-1)
    return out.reshape(b, s, d)


# speedup vs baseline: 1.1818x; 1.1818x over previous
import functools

import jax
import jax.numpy as jnp
from jax import lax
from jax.experimental import pallas as pl
from jax.experimental.pallas import tpu as pltpu

_UNROLL = 16
_T_TILE = 1024


def _round_up(x, m):
    return (x + m - 1) // m * m


def _gather_kernel(idx_ref, table_hbm, out_ref, tab_vmem, sem, *, t_tile, unroll):
    t = pl.program_id(1)

    @pl.when(t == 0)
    def _load_table_half():
        dd = pl.program_id(0)
        cp = pltpu.make_async_copy(
            table_hbm.at[:, pl.ds(dd, 1), :], tab_vmem, sem)
        cp.start()
        cp.wait()

    base = t * t_tile

    def chunk(cb, carry):
        t0 = cb * unroll
        for u in range(unroll):
            loc = t0 + u
            row = idx_ref[base + loc]
            out_ref[loc, 0] = tab_vmem[row, 0]
        return carry

    lax.fori_loop(0, t_tile // unroll, chunk, 0)


def _interleave_kernel(h_ref, out_ref, *, d_half):
    out_ref[:, :d_half] = h_ref[0]
    out_ref[:, d_half:] = h_ref[1]


def kernel(indices, table):
    b, s = indices.shape
    v, d = table.shape
    n_tok = b * s
    d_half = d // 2

    flat_idx = jnp.clip(indices.reshape(-1).astype(jnp.int32), 0, v - 1)

    t_tile = min(_T_TILE, _round_up(n_tok, _UNROLL))
    n_pad = _round_up(n_tok, t_tile)
    if n_pad != n_tok:
        flat_idx = jnp.pad(flat_idx, (0, n_pad - n_tok))
    n_tiles = n_pad // t_tile

    table_r = table.reshape(v, 2, d_half)

    grid_spec = pltpu.PrefetchScalarGridSpec(
        num_scalar_prefetch=1,
        grid=(2, n_tiles),
        in_specs=[pl.BlockSpec(memory_space=pl.ANY)],
        out_specs=pl.BlockSpec(
            (t_tile, 1, d_half), lambda dd, t, idx: (dd * n_tiles + t, 0, 0)),
        scratch_shapes=[
            pltpu.VMEM((v, 1, d_half), table.dtype),
            pltpu.SemaphoreType.DMA,
        ],
    )

    table_half_bytes = v * d_half * jnp.dtype(table.dtype).itemsize
    tile_bytes = t_tile * d_half * jnp.dtype(table.dtype).itemsize
    vmem_limit = int(min(table_half_bytes + 4 * tile_bytes + (8 << 20), 62 << 20))

    halves = pl.pallas_call(
        functools.partial(_gather_kernel, t_tile=t_tile, unroll=_UNROLL),
        out_shape=jax.ShapeDtypeStruct((2 * n_pad, 1, d_half), table.dtype),
        grid_spec=grid_spec,
        compiler_params=pltpu.CompilerParams(
            dimension_semantics=("parallel", "arbitrary"),
            vmem_limit_bytes=vmem_limit,
        ),
    )(flat_idx, table_r)

    halves = halves.reshape(2, n_pad, d_half)

    it_tile = min(512, t_tile)
    out = pl.pallas_call(
        functools.partial(_interleave_kernel, d_half=d_half),
        out_shape=jax.ShapeDtypeStruct((n_pad, d), table.dtype),
        grid=(n_pad // it_tile,),
        in_specs=[pl.BlockSpec((2, it_tile, d_half), lambda t: (0, t, 0))],
        out_specs=pl.BlockSpec((it_tile, d), lambda t: (t, 0)),
        compiler_params=pltpu.CompilerParams(
            dimension_semantics=("parallel",),
        ),
    )(halves)

    return out[:n_tok].reshape(b, s, d)


# speedup vs baseline: 3.0873x; 2.6123x over previous
import functools

import jax
import jax.numpy as jnp
from jax import lax
from jax.experimental import pallas as pl
from jax.experimental.pallas import tpu as pltpu

_UNROLL = 16
_T_TILE = 1024


def _round_up(x, m):
    return (x + m - 1) // m * m


def _gather_kernel(idx_ref, table_hbm, out_ref, tab_vmem, sem,
                   *, t_tile, unroll, d_half):
    t = pl.program_id(1)

    @pl.when(t == 0)
    def _load_table_half():
        dd = pl.program_id(0)
        cp = pltpu.make_async_copy(
            table_hbm.at[:, pl.ds(dd * d_half, d_half)],
            tab_vmem.at[:, 0, :],
            sem)
        cp.start()
        cp.wait()

    base = t * t_tile

    def chunk(cb, carry):
        t0 = cb * unroll
        for u in range(unroll):
            loc = t0 + u
            row = idx_ref[base + loc]
            out_ref[loc, 0] = tab_vmem[row, 0]
        return carry

    lax.fori_loop(0, t_tile // unroll, chunk, 0)


def kernel(indices, table):
    b, s = indices.shape
    v, d = table.shape
    n_tok = b * s
    d_half = d // 2

    flat_idx = jnp.clip(indices.reshape(-1).astype(jnp.int32), 0, v - 1)

    t_tile = min(_T_TILE, _round_up(n_tok, _UNROLL))
    n_pad = _round_up(n_tok, t_tile)
    if n_pad != n_tok:
        flat_idx = jnp.pad(flat_idx, (0, n_pad - n_tok))
    n_tiles = n_pad // t_tile

    grid_spec = pltpu.PrefetchScalarGridSpec(
        num_scalar_prefetch=1,
        grid=(2, n_tiles),
        in_specs=[pl.BlockSpec(memory_space=pl.ANY)],
        out_specs=pl.BlockSpec(
            (t_tile, 1, d_half), lambda dd, t, idx: (t, 0, dd)),
        scratch_shapes=[
            pltpu.VMEM((v, 1, d_half), table.dtype),
            pltpu.SemaphoreType.DMA,
        ],
    )

    table_half_bytes = v * d_half * jnp.dtype(table.dtype).itemsize
    tile_bytes = t_tile * d_half * jnp.dtype(table.dtype).itemsize
    vmem_limit = int(min(table_half_bytes + 4 * tile_bytes + (8 << 20), 62 << 20))

    out = pl.pallas_call(
        functools.partial(_gather_kernel, t_tile=t_tile, unroll=_UNROLL,
                          d_half=d_half),
        out_shape=jax.ShapeDtypeStruct((n_pad, 1, d), table.dtype),
        grid_spec=grid_spec,
        compiler_params=pltpu.CompilerParams(
            dimension_semantics=("parallel", "arbitrary"),
            vmem_limit_bytes=vmem_limit,
        ),
    )(flat_idx, table)

    return out[:n_tok].reshape(b, s, d)


# speedup vs baseline: 4.1615x; 1.3479x over previous
import functools

import jax
import jax.numpy as jnp
from jax import lax
from jax.experimental import pallas as pl
from jax.experimental.pallas import tpu as pltpu

_UNROLL = 16
_T_TILE = 1024


def _round_up(x, m):
    return (x + m - 1) // m * m


def _gather_kernel(idx_ref, table_hbm, out_ref, tab_vmem, sem,
                   *, t_tile, unroll, d_half):
    t = pl.program_id(1)

    @pl.when(t == 0)
    def _load_table_half():
        dd = pl.program_id(0)
        cp = pltpu.make_async_copy(
            table_hbm.at[:, pl.ds(dd * d_half, d_half)],
            tab_vmem.at[:, 0, :],
            sem)
        cp.start()
        cp.wait()

    base = t * t_tile

    def chunk(cb, carry):
        t0 = cb * unroll
        for g in range(unroll // 8):
            rows = []
            for u in range(8):
                loc = t0 + g * 8 + u
                row = idx_ref[base + loc]
                rows.append(tab_vmem[row, 0])
            off = pl.multiple_of(t0 + g * 8, 8)
            out_ref[pl.ds(off, 8), :] = jnp.stack(rows, axis=0)
        return carry

    lax.fori_loop(0, t_tile // unroll, chunk, 0)


def kernel(indices, table):
    b, s = indices.shape
    v, d = table.shape
    n_tok = b * s
    d_half = d // 2

    flat_idx = jnp.clip(indices.reshape(-1).astype(jnp.int32), 0, v - 1)

    t_tile = min(_T_TILE, _round_up(n_tok, _UNROLL))
    n_pad = _round_up(n_tok, t_tile)
    if n_pad != n_tok:
        flat_idx = jnp.pad(flat_idx, (0, n_pad - n_tok))
    n_tiles = n_pad // t_tile

    grid_spec = pltpu.PrefetchScalarGridSpec(
        num_scalar_prefetch=1,
        grid=(2, n_tiles),
        in_specs=[pl.BlockSpec(memory_space=pl.ANY)],
        out_specs=pl.BlockSpec(
            (t_tile, d_half), lambda dd, t, idx: (t, dd)),
        scratch_shapes=[
            pltpu.VMEM((v, 1, d_half), table.dtype),
            pltpu.SemaphoreType.DMA,
        ],
    )

    table_half_bytes = v * d_half * jnp.dtype(table.dtype).itemsize
    tile_bytes = t_tile * d_half * jnp.dtype(table.dtype).itemsize
    vmem_limit = int(min(table_half_bytes + 4 * tile_bytes + (8 << 20), 62 << 20))

    out = pl.pallas_call(
        functools.partial(_gather_kernel, t_tile=t_tile, unroll=_UNROLL,
                          d_half=d_half),
        out_shape=jax.ShapeDtypeStruct((n_pad, d), table.dtype),
        grid_spec=grid_spec,
        compiler_params=pltpu.CompilerParams(
            dimension_semantics=("parallel", "arbitrary"),
            vmem_limit_bytes=vmem_limit,
        ),
    )(flat_idx, table)

    return out[:n_tok].reshape(b, s, d)


# speedup vs baseline: 4.4668x; 1.0734x over previous
import functools

import jax
import jax.numpy as jnp
from jax import lax
from jax.experimental import pallas as pl
from jax.experimental.pallas import tpu as pltpu

_UNROLL = 32
_T_TILE = 2048


def _round_up(x, m):
    return (x + m - 1) // m * m


def _gather_kernel(idx_ref, table_hbm, out_ref, tab_vmem, sem,
                   *, t_tile, unroll, d_half):
    t = pl.program_id(1)

    @pl.when(t == 0)
    def _load_table_half():
        dd = pl.program_id(0)
        cp = pltpu.make_async_copy(
            table_hbm.at[:, pl.ds(dd * d_half, d_half)],
            tab_vmem.at[:, 0, :],
            sem)
        cp.start()
        cp.wait()

    base = t * t_tile

    def chunk(cb, carry):
        t0 = cb * unroll
        for g in range(unroll // 8):
            rows = []
            for u in range(8):
                loc = t0 + g * 8 + u
                row = idx_ref[base + loc]
                rows.append(tab_vmem[row, 0])
            off = pl.multiple_of(t0 + g * 8, 8)
            out_ref[pl.ds(off, 8), :] = jnp.stack(rows, axis=0)
        return carry

    lax.fori_loop(0, t_tile // unroll, chunk, 0)


def kernel(indices, table):
    b, s = indices.shape
    v, d = table.shape
    n_tok = b * s
    d_half = d // 2

    flat_idx = jnp.clip(indices.reshape(-1).astype(jnp.int32), 0, v - 1)

    t_tile = min(_T_TILE, _round_up(n_tok, _UNROLL))
    n_pad = _round_up(n_tok, t_tile)
    if n_pad != n_tok:
        flat_idx = jnp.pad(flat_idx, (0, n_pad - n_tok))
    n_tiles = n_pad // t_tile

    grid_spec = pltpu.PrefetchScalarGridSpec(
        num_scalar_prefetch=1,
        grid=(2, n_tiles),
        in_specs=[pl.BlockSpec(memory_space=pl.ANY)],
        out_specs=pl.BlockSpec(
            (t_tile, d_half), lambda dd, t, idx: (t, dd)),
        scratch_shapes=[
            pltpu.VMEM((v, 1, d_half), table.dtype),
            pltpu.SemaphoreType.DMA,
        ],
    )

    table_half_bytes = v * d_half * jnp.dtype(table.dtype).itemsize
    tile_bytes = t_tile * d_half * jnp.dtype(table.dtype).itemsize
    vmem_limit = int(min(table_half_bytes + 4 * tile_bytes + (8 << 20), 62 << 20))

    out = pl.pallas_call(
        functools.partial(_gather_kernel, t_tile=t_tile, unroll=_UNROLL,
                          d_half=d_half),
        out_shape=jax.ShapeDtypeStruct((n_pad, d), table.dtype),
        grid_spec=grid_spec,
        compiler_params=pltpu.CompilerParams(
            dimension_semantics=("parallel", "arbitrary"),
            vmem_limit_bytes=vmem_limit,
        ),
    )(flat_idx, table)

    return out[:n_tok].reshape(b, s, d)


# speedup vs baseline: 4.4702x; 1.0008x over previous
import functools

import jax
import jax.numpy as jnp
from jax import lax
from jax.experimental import pallas as pl
from jax.experimental.pallas import tpu as pltpu

_UNROLL = 32
_T_TILE = 2048


def _round_up(x, m):
    return (x + m - 1) // m * m


def _gather_kernel(idx_ref, table_hbm, out_ref, tab_vmem, sem,
                   *, t_tile, unroll, d_half):
    t = pl.program_id(1)

    @pl.when(t == 0)
    def _load_table_half():
        dd = pl.program_id(0)
        cp = pltpu.make_async_copy(
            table_hbm.at[:, pl.ds(dd * d_half, d_half)],
            tab_vmem.at[:, 0, :],
            sem)
        cp.start()
        cp.wait()

    base = t * t_tile

    def chunk(cb, carry):
        t0 = cb * unroll
        for g in range(unroll // 8):
            rows = []
            for u in range(8):
                loc = t0 + g * 8 + u
                row = idx_ref[base + loc]
                rows.append(tab_vmem[row, 0])
            off = pl.multiple_of(t0 + g * 8, 8)
            out_ref[pl.ds(off, 8), :] = jnp.stack(rows, axis=0)
        return carry

    lax.fori_loop(0, t_tile // unroll, chunk, 0)


def kernel(indices, table):
    b, s = indices.shape
    v, d = table.shape
    n_tok = b * s
    d_half = d // 2

    flat_idx = indices.reshape(-1).astype(jnp.int32)

    t_tile = min(_T_TILE, _round_up(n_tok, _UNROLL))
    n_pad = _round_up(n_tok, t_tile)
    if n_pad != n_tok:
        flat_idx = jnp.pad(flat_idx, (0, n_pad - n_tok))
    n_tiles = n_pad // t_tile

    grid_spec = pltpu.PrefetchScalarGridSpec(
        num_scalar_prefetch=1,
        grid=(2, n_tiles),
        in_specs=[pl.BlockSpec(memory_space=pl.ANY)],
        out_specs=pl.BlockSpec(
            (t_tile, d_half), lambda dd, t, idx: (t, dd)),
        scratch_shapes=[
            pltpu.VMEM((v, 1, d_half), table.dtype),
            pltpu.SemaphoreType.DMA,
        ],
    )

    table_half_bytes = v * d_half * jnp.dtype(table.dtype).itemsize
    tile_bytes = t_tile * d_half * jnp.dtype(table.dtype).itemsize
    vmem_limit = int(min(table_half_bytes + 4 * tile_bytes + (8 << 20), 62 << 20))

    out = pl.pallas_call(
        functools.partial(_gather_kernel, t_tile=t_tile, unroll=_UNROLL,
                          d_half=d_half),
        out_shape=jax.ShapeDtypeStruct((n_pad, d), table.dtype),
        grid_spec=grid_spec,
        compiler_params=pltpu.CompilerParams(
            dimension_semantics=("parallel", "arbitrary"),
            vmem_limit_bytes=vmem_limit,
        ),
    )(flat_idx, table)

    return out[:n_tok].reshape(b, s, d)


# speedup vs baseline: 4.5017x; 1.0070x over previous
import functools

import jax
import jax.numpy as jnp
from jax import lax
from jax.experimental import pallas as pl
from jax.experimental.pallas import tpu as pltpu

_UNROLL = 64
_T_TILE = 4096


def _round_up(x, m):
    return (x + m - 1) // m * m


def _gather_kernel(idx_ref, table_hbm, out_ref, tab_vmem, sem,
                   *, t_tile, unroll, d_half):
    t = pl.program_id(1)

    @pl.when(t == 0)
    def _load_table_half():
        dd = pl.program_id(0)
        cp = pltpu.make_async_copy(
            table_hbm.at[:, pl.ds(dd * d_half, d_half)],
            tab_vmem.at[:, 0, :],
            sem)
        cp.start()
        cp.wait()

    base = t * t_tile

    def chunk(cb, carry):
        t0 = cb * unroll
        for g in range(unroll // 8):
            rows = []
            for u in range(8):
                loc = t0 + g * 8 + u
                row = idx_ref[base + loc]
                rows.append(tab_vmem[row, 0])
            off = pl.multiple_of(t0 + g * 8, 8)
            out_ref[pl.ds(off, 8), :] = jnp.stack(rows, axis=0)
        return carry

    lax.fori_loop(0, t_tile // unroll, chunk, 0)


def kernel(indices, table):
    b, s = indices.shape
    v, d = table.shape
    n_tok = b * s
    d_half = d // 2

    flat_idx = indices.reshape(-1).astype(jnp.int32)

    t_tile = min(_T_TILE, _round_up(n_tok, _UNROLL))
    n_pad = _round_up(n_tok, t_tile)
    if n_pad != n_tok:
        flat_idx = jnp.pad(flat_idx, (0, n_pad - n_tok))
    n_tiles = n_pad // t_tile

    grid_spec = pltpu.PrefetchScalarGridSpec(
        num_scalar_prefetch=1,
        grid=(2, n_tiles),
        in_specs=[pl.BlockSpec(memory_space=pl.ANY)],
        out_specs=pl.BlockSpec(
            (t_tile, d_half), lambda dd, t, idx: (t, dd)),
        scratch_shapes=[
            pltpu.VMEM((v, 1, d_half), table.dtype),
            pltpu.SemaphoreType.DMA,
        ],
    )

    table_half_bytes = v * d_half * jnp.dtype(table.dtype).itemsize
    tile_bytes = t_tile * d_half * jnp.dtype(table.dtype).itemsize
    vmem_limit = int(min(table_half_bytes + 4 * tile_bytes + (8 << 20), 62 << 20))

    out = pl.pallas_call(
        functools.partial(_gather_kernel, t_tile=t_tile, unroll=_UNROLL,
                          d_half=d_half),
        out_shape=jax.ShapeDtypeStruct((n_pad, d), table.dtype),
        grid_spec=grid_spec,
        compiler_params=pltpu.CompilerParams(
            dimension_semantics=("parallel", "arbitrary"),
            vmem_limit_bytes=vmem_limit,
        ),
    )(flat_idx, table)

    return out[:n_tok].reshape(b, s, d)
